# Initial kernel scaffold; baseline (speedup 1.0000x reference)
import jax
import jax.numpy as jnp
from jax.experimental import pallas as pl


def kernel(x, Wg, W1, b1, W2, b2):
    raise NotImplementedError("write your pallas kernel here")


# speedup vs baseline: 2.1101x; 2.1101x over previous
import functools

import jax
import jax.numpy as jnp
from jax.experimental import pallas as pl

_IN = 768
_E = 8
_D = 128
_B = 2048
_TB = 256


def _moe_body(x_ref, wgt_ref, w1t_ref, b1_ref, w2t_ref, b2_ref,
              out_ref, gw_ref):
    x = x_ref[...]
    logits = jnp.dot(x, wgt_ref[...],
                     preferred_element_type=jnp.float32)
    m1 = jnp.max(logits, axis=-1, keepdims=True)
    masked = jnp.where(logits == m1, -jnp.inf, logits)
    m2 = jnp.max(masked, axis=-1, keepdims=True)
    denom = 1.0 + jnp.exp(m2 - m1)
    gw = jnp.where(logits >= m2, jnp.exp(logits - m1), 0.0) / denom
    gw_ref[...] = gw

    h = jnp.dot(x, w1t_ref[...],
                preferred_element_type=jnp.float32) + b1_ref[...]
    h = 0.5 * h * (1.0 + jax.lax.erf(h * 0.7071067811865476))
    hw = (h.reshape(_TB, _E, _D) * gw[:, :, None]).reshape(_TB, _E * _D)
    out = jnp.dot(hw, w2t_ref[...], preferred_element_type=jnp.float32)
    out_ref[...] = out + jnp.dot(gw, b2_ref[...],
                                 preferred_element_type=jnp.float32)


@jax.jit
def kernel(x, Wg, W1, b1, W2, b2):
    wgt = Wg.T
    w1t = W1.reshape(_E * _D, _IN).T
    b1f = b1.reshape(1, _E * _D)
    w2t = jnp.transpose(W2, (0, 2, 1)).reshape(_E * _D, _D)
    grid = (_B // _TB,)
    const = lambda i: (0, 0)
    out, gw = pl.pallas_call(
        _moe_body,
        grid=grid,
        in_specs=[
            pl.BlockSpec((_TB, _IN), lambda i: (i, 0)),
            pl.BlockSpec((_IN, _E), const),
            pl.BlockSpec((_IN, _E * _D), const),
            pl.BlockSpec((1, _E * _D), const),
            pl.BlockSpec((_E * _D, _D), const),
            pl.BlockSpec((_E, _D), const),
        ],
        out_specs=[
            pl.BlockSpec((_TB, _D), lambda i: (i, 0)),
            pl.BlockSpec((_TB, _E), lambda i: (i, 0)),
        ],
        out_shape=[
            jax.ShapeDtypeStruct((_B, _D), jnp.float32),
            jax.ShapeDtypeStruct((_B, _E), jnp.float32),
        ],
    )(x, wgt, w1t, b1f, w2t, b2)
    return out, gw


# speedup vs baseline: 2.4780x; 1.1744x over previous
import jax
import jax.numpy as jnp
from jax.experimental import pallas as pl

_IN = 768
_E = 8
_D = 128
_B = 2048
_TB = 256


def _moe_body(x_ref, wg_ref, w1_ref, b1_ref, w2t_ref, b2_ref,
              out_ref, gw_ref):
    x = x_ref[...]
    logits = jax.lax.dot_general(
        x, wg_ref[...], (((1,), (1,)), ((), ())),
        preferred_element_type=jnp.float32)
    m1 = jnp.max(logits, axis=-1, keepdims=True)
    masked = jnp.where(logits == m1, -jnp.inf, logits)
    m2 = jnp.max(masked, axis=-1, keepdims=True)
    denom = 1.0 + jnp.exp(m2 - m1)
    gw = jnp.where(logits >= m2, jnp.exp(logits - m1), 0.0) / denom
    gw_ref[...] = gw

    xb = x.astype(jnp.bfloat16)
    h = jax.lax.dot_general(
        xb, w1_ref[...], (((1,), (1,)), ((), ())),
        preferred_element_type=jnp.float32) + b1_ref[...]
    h = 0.5 * h * (1.0 + jax.lax.erf(h * 0.7071067811865476))
    hw = ((h.reshape(_TB, _E, _D) * gw[:, :, None])
          .reshape(_TB, _E * _D).astype(jnp.bfloat16))
    out = jnp.dot(hw, w2t_ref[...], preferred_element_type=jnp.float32)
    out_ref[...] = out + jnp.dot(gw, b2_ref[...],
                                 preferred_element_type=jnp.float32)


@jax.jit
def kernel(x, Wg, W1, b1, W2, b2):
    w1b = W1.reshape(_E * _D, _IN).astype(jnp.bfloat16)
    b1f = b1.reshape(1, _E * _D)
    w2t = (jnp.transpose(W2, (0, 2, 1)).reshape(_E * _D, _D)
           .astype(jnp.bfloat16))
    grid = (_B // _TB,)
    const = lambda i: (0, 0)
    out, gw = pl.pallas_call(
        _moe_body,
        grid=grid,
        in_specs=[
            pl.BlockSpec((_TB, _IN), lambda i: (i, 0)),
            pl.BlockSpec((_E, _IN), const),
            pl.BlockSpec((_E * _D, _IN), const),
            pl.BlockSpec((1, _E * _D), const),
            pl.BlockSpec((_E * _D, _D), const),
            pl.BlockSpec((_E, _D), const),
        ],
        out_specs=[
            pl.BlockSpec((_TB, _D), lambda i: (i, 0)),
            pl.BlockSpec((_TB, _E), lambda i: (i, 0)),
        ],
        out_shape=[
            jax.ShapeDtypeStruct((_B, _D), jnp.float32),
            jax.ShapeDtypeStruct((_B, _E), jnp.float32),
        ],
    )(x, Wg, w1b, b1f, w2t, b2)
    return out, gw


# speedup vs baseline: 3.5546x; 1.4345x over previous
import jax
import jax.numpy as jnp
from jax.experimental import pallas as pl
from jax.experimental.pallas import tpu as pltpu

_IN = 768
_E = 8
_D = 128
_B = 2048
_TB = 256


def _moe_body(x_ref, wg_ref, w1_ref, b1_ref, w2_ref, b2_ref,
              out_ref, gw_ref, w1b_ref, w2b_ref, sel_ref):
    i = pl.program_id(0)

    @pl.when(i == 0)
    def _prep():
        w1b_ref[...] = w1_ref[...].astype(jnp.bfloat16)
        w2b_ref[...] = (jnp.transpose(w2_ref[...], (0, 2, 1))
                        .reshape(_E * _D, _D).astype(jnp.bfloat16))
        sel_ref[...] = (
            jax.lax.broadcasted_iota(jnp.int32, (_E, _E * _D), 0)
            == jax.lax.broadcasted_iota(jnp.int32, (_E, _E * _D), 1) // _D
        ).astype(jnp.float32)

    x = x_ref[...]
    logits = jax.lax.dot_general(
        x, wg_ref[...], (((1,), (1,)), ((), ())),
        preferred_element_type=jnp.float32)
    m1 = jnp.max(logits, axis=-1, keepdims=True)
    masked = jnp.where(logits == m1, -jnp.inf, logits)
    m2 = jnp.max(masked, axis=-1, keepdims=True)
    denom = 1.0 + jnp.exp(m2 - m1)
    gw = jnp.where(logits >= m2, jnp.exp(logits - m1), 0.0) / denom
    gw_ref[...] = gw

    xb = x.astype(jnp.bfloat16)
    h = jax.lax.dot_general(
        xb, w1b_ref[...], (((1,), (1,)), ((), ())),
        preferred_element_type=jnp.float32) + b1_ref[...]
    h = 0.5 * h * (1.0 + jax.lax.erf(h * 0.7071067811865476))
    w_rep = jnp.dot(gw, sel_ref[...],
                    preferred_element_type=jnp.float32)
    hw = (h * w_rep).astype(jnp.bfloat16)
    out = jnp.dot(hw, w2b_ref[...], preferred_element_type=jnp.float32)
    out_ref[...] = out + jnp.dot(gw, b2_ref[...],
                                 preferred_element_type=jnp.float32)


@jax.jit
def kernel(x, Wg, W1, b1, W2, b2):
    w1r = W1.reshape(_E * _D, _IN)
    b1f = b1.reshape(1, _E * _D)
    grid = (_B // _TB,)
    const2 = lambda i: (0, 0)
    const3 = lambda i: (0, 0, 0)
    out, gw = pl.pallas_call(
        _moe_body,
        grid=grid,
        in_specs=[
            pl.BlockSpec((_TB, _IN), lambda i: (i, 0)),
            pl.BlockSpec((_E, _IN), const2),
            pl.BlockSpec((_E * _D, _IN), const2),
            pl.BlockSpec((1, _E * _D), const2),
            pl.BlockSpec((_E, _D, _D), const3),
            pl.BlockSpec((_E, _D), const2),
        ],
        out_specs=[
            pl.BlockSpec((_TB, _D), lambda i: (i, 0)),
            pl.BlockSpec((_TB, _E), lambda i: (i, 0)),
        ],
        out_shape=[
            jax.ShapeDtypeStruct((_B, _D), jnp.float32),
            jax.ShapeDtypeStruct((_B, _E), jnp.float32),
        ],
        scratch_shapes=[
            pltpu.VMEM((_E * _D, _IN), jnp.bfloat16),
            pltpu.VMEM((_E * _D, _D), jnp.bfloat16),
            pltpu.VMEM((_E, _E * _D), jnp.float32),
        ],
    )(x, Wg, w1r, b1f, W2, b2)
    return out, gw


# speedup vs baseline: 3.9497x; 1.1112x over previous
import jax
import jax.numpy as jnp
from jax.experimental import pallas as pl
from jax.experimental.pallas import tpu as pltpu

_IN = 768
_E = 8
_D = 128
_B = 2048
_TB = 512


def _moe_body(x_ref, wg_ref, w1_ref, b1_ref, w2_ref, b2_ref,
              out_ref, gw_ref, w1b_ref, w2b_ref, sel_ref):
    i = pl.program_id(0)

    @pl.when(i == 0)
    def _prep():
        w1b_ref[...] = w1_ref[...].astype(jnp.bfloat16)
        w2b_ref[...] = (jnp.transpose(w2_ref[...], (0, 2, 1))
                        .reshape(_E * _D, _D).astype(jnp.bfloat16))
        sel_ref[...] = (
            jax.lax.broadcasted_iota(jnp.int32, (_E, _E * _D), 0)
            == jax.lax.broadcasted_iota(jnp.int32, (_E, _E * _D), 1) // _D
        ).astype(jnp.bfloat16)

    x = x_ref[...]
    logits = jax.lax.dot_general(
        x, wg_ref[...], (((1,), (1,)), ((), ())),
        preferred_element_type=jnp.float32)
    m1 = jnp.max(logits, axis=-1, keepdims=True)
    masked = jnp.where(logits == m1, -jnp.inf, logits)
    m2 = jnp.max(masked, axis=-1, keepdims=True)
    denom = 1.0 + jnp.exp(m2 - m1)
    gw = jnp.where(logits >= m2, jnp.exp(logits - m1), 0.0) / denom
    gw_ref[...] = gw

    xb = x.astype(jnp.bfloat16)
    h = jax.lax.dot_general(
        xb, w1b_ref[...], (((1,), (1,)), ((), ())),
        preferred_element_type=jnp.float32) + b1_ref[...]
    h = 0.5 * h * (1.0 + jax.lax.erf(h * 0.7071067811865476))
    w_rep = jnp.dot(gw.astype(jnp.bfloat16), sel_ref[...],
                    preferred_element_type=jnp.float32)
    hw = (h * w_rep).astype(jnp.bfloat16)
    out = jnp.dot(hw, w2b_ref[...], preferred_element_type=jnp.float32)
    out_ref[...] = out + jnp.dot(gw, b2_ref[...],
                                 preferred_element_type=jnp.float32)


@jax.jit
def kernel(x, Wg, W1, b1, W2, b2):
    w1r = W1.reshape(_E * _D, _IN)
    b1f = b1.reshape(1, _E * _D)
    grid = (_B // _TB,)
    const2 = lambda i: (0, 0)
    const3 = lambda i: (0, 0, 0)
    out, gw = pl.pallas_call(
        _moe_body,
        grid=grid,
        in_specs=[
            pl.BlockSpec((_TB, _IN), lambda i: (i, 0)),
            pl.BlockSpec((_E, _IN), const2),
            pl.BlockSpec((_E * _D, _IN), const2),
            pl.BlockSpec((1, _E * _D), const2),
            pl.BlockSpec((_E, _D, _D), const3),
            pl.BlockSpec((_E, _D), const2),
        ],
        out_specs=[
            pl.BlockSpec((_TB, _D), lambda i: (i, 0)),
            pl.BlockSpec((_TB, _E), lambda i: (i, 0)),
        ],
        out_shape=[
            jax.ShapeDtypeStruct((_B, _D), jnp.float32),
            jax.ShapeDtypeStruct((_B, _E), jnp.float32),
        ],
        scratch_shapes=[
            pltpu.VMEM((_E * _D, _IN), jnp.bfloat16),
            pltpu.VMEM((_E * _D, _D), jnp.bfloat16),
            pltpu.VMEM((_E, _E * _D), jnp.bfloat16),
        ],
    )(x, Wg, w1r, b1f, W2, b2)
    return out, gw


# speedup vs baseline: 3.9653x; 1.0040x over previous
import jax
import jax.numpy as jnp
from jax.experimental import pallas as pl
from jax.experimental.pallas import tpu as pltpu

_IN = 768
_E = 8
_D = 128
_B = 2048
_TB = 1024


def _moe_body(x_ref, wg_ref, w1_ref, b1_ref, w2_ref, b2_ref,
              out_ref, gw_ref, w1b_ref, w2b_ref, sel_ref):
    i = pl.program_id(0)

    @pl.when(i == 0)
    def _prep():
        w1b_ref[...] = w1_ref[...].astype(jnp.bfloat16)
        w2b_ref[...] = (jnp.transpose(w2_ref[...], (0, 2, 1))
                        .reshape(_E * _D, _D).astype(jnp.bfloat16))
        sel_ref[...] = (
            jax.lax.broadcasted_iota(jnp.int32, (_E, _E * _D), 0)
            == jax.lax.broadcasted_iota(jnp.int32, (_E, _E * _D), 1) // _D
        ).astype(jnp.bfloat16)

    x = x_ref[...]
    logits = jax.lax.dot_general(
        x, wg_ref[...], (((1,), (1,)), ((), ())),
        preferred_element_type=jnp.float32)
    m1 = jnp.max(logits, axis=-1, keepdims=True)
    masked = jnp.where(logits == m1, -jnp.inf, logits)
    m2 = jnp.max(masked, axis=-1, keepdims=True)
    denom = 1.0 + jnp.exp(m2 - m1)
    gw = jnp.where(logits >= m2, jnp.exp(logits - m1), 0.0) / denom
    gw_ref[...] = gw

    xb = x.astype(jnp.bfloat16)
    h = jax.lax.dot_general(
        xb, w1b_ref[...], (((1,), (1,)), ((), ())),
        preferred_element_type=jnp.float32) + b1_ref[...]
    h = 0.5 * h * (1.0 + jax.lax.erf(h * 0.7071067811865476))
    w_rep = jnp.dot(gw.astype(jnp.bfloat16), sel_ref[...],
                    preferred_element_type=jnp.float32)
    hw = (h * w_rep).astype(jnp.bfloat16)
    out = jnp.dot(hw, w2b_ref[...], preferred_element_type=jnp.float32)
    out_ref[...] = out + jnp.dot(gw, b2_ref[...],
                                 preferred_element_type=jnp.float32)


@jax.jit
def kernel(x, Wg, W1, b1, W2, b2):
    w1r = W1.reshape(_E * _D, _IN)
    b1f = b1.reshape(1, _E * _D)
    grid = (_B // _TB,)
    const2 = lambda i: (0, 0)
    const3 = lambda i: (0, 0, 0)
    out, gw = pl.pallas_call(
        _moe_body,
        grid=grid,
        in_specs=[
            pl.BlockSpec((_TB, _IN), lambda i: (i, 0)),
            pl.BlockSpec((_E, _IN), const2),
            pl.BlockSpec((_E * _D, _IN), const2),
            pl.BlockSpec((1, _E * _D), const2),
            pl.BlockSpec((_E, _D, _D), const3),
            pl.BlockSpec((_E, _D), const2),
        ],
        out_specs=[
            pl.BlockSpec((_TB, _D), lambda i: (i, 0)),
            pl.BlockSpec((_TB, _E), lambda i: (i, 0)),
        ],
        out_shape=[
            jax.ShapeDtypeStruct((_B, _D), jnp.float32),
            jax.ShapeDtypeStruct((_B, _E), jnp.float32),
        ],
        scratch_shapes=[
            pltpu.VMEM((_E * _D, _IN), jnp.bfloat16),
            pltpu.VMEM((_E * _D, _D), jnp.bfloat16),
            pltpu.VMEM((_E, _E * _D), jnp.bfloat16),
        ],
    )(x, Wg, w1r, b1f, W2, b2)
    return out, gw
